# baseline (speedup 1.0000x reference)
import math
from functools import partial

import jax
import jax.numpy as jnp
import numpy as np
from jax.experimental import pallas as pl
from jax.experimental.pallas import tpu as pltpu

_MASK_PROB = 0.065
_MASK_LENGTH = 10

_I32 = jnp.int32
_F32 = jnp.float32
_SIGN = np.int32(-2147483648)


def _rotl(x, r):
    return jax.lax.shift_left(x, _I32(r)) | jax.lax.shift_right_logical(
        x, _I32(32 - r))


def _threefry2x32(k0, k1, x0, x1):
    ks0, ks1 = k0, k1
    ks2 = ks0 ^ ks1 ^ np.int32(0x1BD11BDA)
    rots = ((13, 15, 26, 6), (17, 29, 16, 24))
    sched = ((ks1, ks2), (ks2, ks0), (ks0, ks1), (ks1, ks2), (ks2, ks0))
    x0 = x0 + ks0
    x1 = x1 + ks1
    for i in range(5):
        for r in rots[i % 2]:
            x0 = x0 + x1
            x1 = _rotl(x1, r)
            x1 = x1 ^ x0
        a, b = sched[i]
        x0 = x0 + a
        x1 = x1 + b + np.int32(i + 1)
    return x0, x1


def _bitonic_sort_pairs(key, idx, flat_iota, rows, cols, total):
    size = 2
    while size <= total:
        d = size // 2
        while d >= 1:
            if d < cols:
                fwd_k = pltpu.roll(key, cols - d, 2)
                bwd_k = pltpu.roll(key, d, 2)
                fwd_i = pltpu.roll(idx, cols - d, 2)
                bwd_i = pltpu.roll(idx, d, 2)
            else:
                dr = d // cols
                fwd_k = pltpu.roll(key, rows - dr, 1)
                bwd_k = pltpu.roll(key, dr, 1)
                fwd_i = pltpu.roll(idx, rows - dr, 1)
                bwd_i = pltpu.roll(idx, dr, 1)
            first = (flat_iota & _I32(d)) == 0
            pk = jnp.where(first, fwd_k, bwd_k)
            pi = jnp.where(first, fwd_i, bwd_i)
            want_min = first == ((flat_iota & _I32(size)) == 0)
            new_key = jnp.where(want_min, jnp.minimum(key, pk),
                                jnp.maximum(key, pk))
            take = new_key != key
            key = new_key
            idx = jnp.where(take, pi, idx)
            d //= 2
        size *= 2
    return key, idx


def _mask_kernel(icol_ref, o_ref, *, nb, rows, cols, n_valid, n_take,
                 num_rounds, mask_len):
    total = rows * cols
    flat_iota = (
        jax.lax.broadcasted_iota(_I32, (nb, rows, cols), 1) * _I32(cols)
        + jax.lax.broadcasted_iota(_I32, (nb, rows, cols), 2))
    brow = jax.lax.broadcasted_iota(_I32, (nb, rows, cols), 0)
    imax = np.int32(2147483647)

    rk0, rk1 = _threefry2x32(_I32(0), _I32(42), _I32(0), brow)
    zero = jnp.zeros_like(flat_iota)
    round_sort_keys = []
    for _ in range(num_rounds):
        nk0, nk1 = _threefry2x32(rk0, rk1, zero, zero)
        sk0, sk1 = _threefry2x32(rk0, rk1, zero, jnp.ones_like(flat_iota))
        o0, o1 = _threefry2x32(sk0, sk1, zero, flat_iota)
        srt = (o0 ^ o1) ^ _SIGN
        round_sort_keys.append(
            jnp.where(flat_iota < _I32(n_valid), srt,
                      jnp.full_like(flat_iota, imax)))
        rk0, rk1 = nk0, nk1
    k1s = round_sort_keys[0]
    k2s = round_sort_keys[-1]

    ones_col = jnp.ones((cols, 1), _F32)

    def _descend54(j, v):
        bit = jax.lax.shift_left(_I32(1), _I32(31) - j)
        try_pat = v | bit
        try_s = try_pat ^ _SIGN
        ltf = jnp.where(k2s < try_s, _F32(1), _F32(0)).reshape(
            nb * rows, cols)
        c1 = jax.lax.dot_general(
            ltf, ones_col, dimension_numbers=(((1,), (0,)), ((), ())),
            preferred_element_type=_F32)
        cnt = jnp.sum(c1.reshape(nb, rows, 1), axis=1,
                      keepdims=True)
        return jnp.where(cnt >= _F32(n_take), v, try_pat)

    v54 = jax.lax.fori_loop(0, 32, _descend54, jnp.zeros((nb, 1, 1), _I32))
    v54_s = v54 ^ _SIGN
    p_ind = jnp.where(k2s <= v54_s, _F32(1), _F32(0))

    _, sidx = _bitonic_sort_pairs(k1s, flat_iota, flat_iota, rows, cols, total)

    ci = jax.lax.broadcasted_iota(_I32, (cols, cols), 0)
    cj = jax.lax.broadcasted_iota(_I32, (cols, cols), 1)
    d_in = cj - ci
    m_in = jnp.where((d_in >= 0) & (d_in < _I32(mask_len)), _F32(1), _F32(0))
    d_x = cj + _I32(cols) - ci
    m_x = jnp.where((d_x >= 0) & (d_x < _I32(mask_len)), _F32(1), _F32(0))
    icol = icol_ref[:, :]
    ih_col = icol[:rows, :]

    for r in range(nb):
        s_flat = sidx[r].reshape(1, total)
        p_flat = p_ind[r].reshape(1, total)
        vh = jax.lax.shift_right_logical(s_flat, _I32(7)).astype(_F32)
        vl = (s_flat & _I32(cols - 1)).astype(_F32)
        s1t = jnp.where((vh == ih_col) & (p_flat > _F32(0)), _F32(1), _F32(0))
        s2t = jnp.where(vl == icol, _F32(1), _F32(0))
        sel = jax.lax.dot_general(
            s1t, s2t, dimension_numbers=(((1,), (1,)), ((), ())),
            preferred_element_type=_F32)
        hit = jax.lax.dot_general(sel, m_in,
                                  dimension_numbers=(((1,), (0,)), ((), ())),
                                  preferred_element_type=_F32)
        sel_prev = jnp.concatenate(
            [jnp.zeros((1, cols), _F32), sel[:rows - 1, :]], axis=0)
        hit = hit + jax.lax.dot_general(
            sel_prev, m_x, dimension_numbers=(((1,), (0,)), ((), ())),
            preferred_element_type=_F32)
        o_ref[r, :, :] = (hit > _F32(0)).astype(jnp.int8)


@jax.jit
def kernel(x):
    B, T, C = x.shape
    total_masked_length = int(T * _MASK_PROB)
    num_masks = math.ceil(total_masked_length / _MASK_LENGTH)
    valid_starts = T - _MASK_LENGTH + 1
    if valid_starts <= 0:
        return jnp.zeros((B, T), dtype=bool)
    n_take = min(num_masks, valid_starts)
    num_rounds = int(
        np.ceil(3 * np.log(max(1, valid_starts)) / np.log(2**32 - 1)))
    cols = 128
    rows = T // cols

    out = pl.pallas_call(
        partial(_mask_kernel, nb=B, rows=rows, cols=cols, n_valid=valid_starts,
                n_take=n_take, num_rounds=num_rounds, mask_len=_MASK_LENGTH),
        grid=(1,),
        in_specs=[pl.BlockSpec((cols, 1), lambda b: (0, 0))],
        out_specs=pl.BlockSpec((B, rows, cols), lambda b: (0, 0, 0)),
        out_shape=jax.ShapeDtypeStruct((B, rows, cols), jnp.int8),
    )(jnp.arange(cols, dtype=jnp.float32).reshape(cols, 1))
    return out.reshape(B, T).astype(bool)


# speedup vs baseline: 1.1535x; 1.1535x over previous
import math
from functools import partial

import jax
import jax.numpy as jnp
import numpy as np
from jax.experimental import pallas as pl
from jax.experimental.pallas import tpu as pltpu

_MASK_PROB = 0.065
_MASK_LENGTH = 10

_I32 = jnp.int32
_F32 = jnp.float32
_SIGN = np.int32(-2147483648)


def _rotl(x, r):
    return jax.lax.shift_left(x, _I32(r)) | jax.lax.shift_right_logical(
        x, _I32(32 - r))


def _threefry2x32(k0, k1, x0, x1):
    ks0, ks1 = k0, k1
    ks2 = ks0 ^ ks1 ^ np.int32(0x1BD11BDA)
    rots = ((13, 15, 26, 6), (17, 29, 16, 24))
    sched = ((ks1, ks2), (ks2, ks0), (ks0, ks1), (ks1, ks2), (ks2, ks0))
    x0 = x0 + ks0
    x1 = x1 + ks1
    for i in range(5):
        for r in rots[i % 2]:
            x0 = x0 + x1
            x1 = _rotl(x1, r)
            x1 = x1 ^ x0
        a, b = sched[i]
        x0 = x0 + a
        x1 = x1 + b + np.int32(i + 1)
    return x0, x1


def _bitonic_sort_pairs(key, idx, flat_iota, rows, cols, total):
    size = 2
    while size <= total:
        d = size // 2
        while d >= 1:
            if d < cols:
                fwd_k = pltpu.roll(key, cols - d, 2)
                bwd_k = pltpu.roll(key, d, 2)
                fwd_i = pltpu.roll(idx, cols - d, 2)
                bwd_i = pltpu.roll(idx, d, 2)
            else:
                dr = d // cols
                fwd_k = pltpu.roll(key, rows - dr, 1)
                bwd_k = pltpu.roll(key, dr, 1)
                fwd_i = pltpu.roll(idx, rows - dr, 1)
                bwd_i = pltpu.roll(idx, dr, 1)
            first = (flat_iota & _I32(d)) == 0
            pk = jnp.where(first, fwd_k, bwd_k)
            pi = jnp.where(first, fwd_i, bwd_i)
            want_min = first == ((flat_iota & _I32(size)) == 0)
            new_key = jnp.where(want_min, jnp.minimum(key, pk),
                                jnp.maximum(key, pk))
            take = new_key != key
            key = new_key
            idx = jnp.where(take, pi, idx)
            d //= 2
        size *= 2
    return key, idx


def _mask_kernel(icol_ref, o_ref, *, nb, rows, cols, n_valid, n_take,
                 num_rounds, mask_len):
    total = rows * cols
    flat_iota = (
        jax.lax.broadcasted_iota(_I32, (nb, rows, cols), 1) * _I32(cols)
        + jax.lax.broadcasted_iota(_I32, (nb, rows, cols), 2))
    brow = jax.lax.broadcasted_iota(_I32, (nb, rows, cols), 0)
    imax = np.int32(2147483647)

    rk0, rk1 = _threefry2x32(_I32(0), _I32(42), _I32(0), brow)
    zero = jnp.zeros_like(flat_iota)
    round_sort_keys = []
    for _ in range(num_rounds):
        nk0, nk1 = _threefry2x32(rk0, rk1, zero, zero)
        sk0, sk1 = _threefry2x32(rk0, rk1, zero, jnp.ones_like(flat_iota))
        o0, o1 = _threefry2x32(sk0, sk1, zero, flat_iota)
        srt = (o0 ^ o1) ^ _SIGN
        round_sort_keys.append(
            jnp.where(flat_iota < _I32(n_valid), srt,
                      jnp.full_like(flat_iota, imax)))
        rk0, rk1 = nk0, nk1
    k1s = round_sort_keys[0]
    k2s = round_sort_keys[-1]

    def _descend54(j, v):
        bit = jax.lax.shift_left(_I32(1), _I32(31) - j)
        try_pat = v | bit
        try_s = try_pat ^ _SIGN
        cnt = jnp.sum(jnp.where(k2s < try_s, _F32(1), _F32(0)),
                      axis=(1, 2), keepdims=True)
        return jnp.where(cnt >= _F32(n_take), v, try_pat)

    v54 = jax.lax.fori_loop(0, 32, _descend54, jnp.zeros((nb, 1, 1), _I32))
    v54_s = v54 ^ _SIGN
    p_ind = jnp.where(k2s <= v54_s, _F32(1), _F32(0))

    _, sidx = _bitonic_sort_pairs(k1s, flat_iota, flat_iota, rows, cols, total)

    ci = jax.lax.broadcasted_iota(_I32, (cols, cols), 0)
    cj = jax.lax.broadcasted_iota(_I32, (cols, cols), 1)
    d_in = cj - ci
    m_in = jnp.where((d_in >= 0) & (d_in < _I32(mask_len)), _F32(1), _F32(0))
    d_x = cj + _I32(cols) - ci
    m_x = jnp.where((d_x >= 0) & (d_x < _I32(mask_len)), _F32(1), _F32(0))
    icol = icol_ref[:, :]
    ih_col = icol[:rows, :]

    for r in range(nb):
        s_flat = sidx[r].reshape(1, total)
        p_flat = p_ind[r].reshape(1, total)
        vh = jax.lax.shift_right_logical(s_flat, _I32(7)).astype(_F32)
        vl = (s_flat & _I32(cols - 1)).astype(_F32)
        s1t = jnp.where((vh == ih_col) & (p_flat > _F32(0)), _F32(1), _F32(0))
        s2t = jnp.where(vl == icol, _F32(1), _F32(0))
        sel = jax.lax.dot_general(
            s1t, s2t, dimension_numbers=(((1,), (1,)), ((), ())),
            preferred_element_type=_F32)
        hit = jax.lax.dot_general(sel, m_in,
                                  dimension_numbers=(((1,), (0,)), ((), ())),
                                  preferred_element_type=_F32)
        sel_prev = jnp.concatenate(
            [jnp.zeros((1, cols), _F32), sel[:rows - 1, :]], axis=0)
        hit = hit + jax.lax.dot_general(
            sel_prev, m_x, dimension_numbers=(((1,), (0,)), ((), ())),
            preferred_element_type=_F32)
        o_ref[r, :, :] = (hit > _F32(0)).astype(jnp.int8)


@jax.jit
def kernel(x):
    B, T, C = x.shape
    total_masked_length = int(T * _MASK_PROB)
    num_masks = math.ceil(total_masked_length / _MASK_LENGTH)
    valid_starts = T - _MASK_LENGTH + 1
    if valid_starts <= 0:
        return jnp.zeros((B, T), dtype=bool)
    n_take = min(num_masks, valid_starts)
    num_rounds = int(
        np.ceil(3 * np.log(max(1, valid_starts)) / np.log(2**32 - 1)))
    cols = 128
    rows = T // cols

    out = pl.pallas_call(
        partial(_mask_kernel, nb=B, rows=rows, cols=cols, n_valid=valid_starts,
                n_take=n_take, num_rounds=num_rounds, mask_len=_MASK_LENGTH),
        grid=(1,),
        in_specs=[pl.BlockSpec((cols, 1), lambda b: (0, 0))],
        out_specs=pl.BlockSpec((B, rows, cols), lambda b: (0, 0, 0)),
        out_shape=jax.ShapeDtypeStruct((B, rows, cols), jnp.int8),
    )(jnp.arange(cols, dtype=jnp.float32).reshape(cols, 1))
    return out.reshape(B, T).astype(bool)


# speedup vs baseline: 1.2810x; 1.1106x over previous
import math
from functools import partial

import jax
import jax.numpy as jnp
import numpy as np
from jax.experimental import pallas as pl
from jax.experimental.pallas import tpu as pltpu

_MASK_PROB = 0.065
_MASK_LENGTH = 10

_I32 = jnp.int32
_F32 = jnp.float32
_SIGN = np.int32(-2147483648)


def _rotl(x, r):
    return jax.lax.shift_left(x, _I32(r)) | jax.lax.shift_right_logical(
        x, _I32(32 - r))


def _threefry2x32(k0, k1, x0, x1):
    ks0, ks1 = k0, k1
    ks2 = ks0 ^ ks1 ^ np.int32(0x1BD11BDA)
    rots = ((13, 15, 26, 6), (17, 29, 16, 24))
    sched = ((ks1, ks2), (ks2, ks0), (ks0, ks1), (ks1, ks2), (ks2, ks0))
    x0 = x0 + ks0
    x1 = x1 + ks1
    for i in range(5):
        for r in rots[i % 2]:
            x0 = x0 + x1
            x1 = _rotl(x1, r)
            x1 = x1 ^ x0
        a, b = sched[i]
        x0 = x0 + a
        x1 = x1 + b + np.int32(i + 1)
    return x0, x1


def _bitonic_sort_pairs(key, idx, flat_iota, rows, cols, total):
    size = 2
    while size <= total:
        d = size // 2
        while d >= 1:
            if d < cols:
                fwd_k = pltpu.roll(key, cols - d, 2)
                bwd_k = pltpu.roll(key, d, 2)
                fwd_i = pltpu.roll(idx, cols - d, 2)
                bwd_i = pltpu.roll(idx, d, 2)
            else:
                dr = d // cols
                fwd_k = pltpu.roll(key, rows - dr, 1)
                bwd_k = pltpu.roll(key, dr, 1)
                fwd_i = pltpu.roll(idx, rows - dr, 1)
                bwd_i = pltpu.roll(idx, dr, 1)
            first = (flat_iota & _I32(d)) == 0
            pk = jnp.where(first, fwd_k, bwd_k)
            pi = jnp.where(first, fwd_i, bwd_i)
            want_min = first == ((flat_iota & _I32(size)) == 0)
            new_key = jnp.where(want_min, jnp.minimum(key, pk),
                                jnp.maximum(key, pk))
            take = new_key != key
            key = new_key
            idx = jnp.where(take, pi, idx)
            d //= 2
        size *= 2
    return key, idx


def _mask_kernel(icol_ref, o_ref, *, nb, rows, cols, n_valid, n_take,
                 num_rounds, mask_len):
    total = rows * cols
    flat_iota = (
        jax.lax.broadcasted_iota(_I32, (nb, rows, cols), 1) * _I32(cols)
        + jax.lax.broadcasted_iota(_I32, (nb, rows, cols), 2))
    brow = jax.lax.broadcasted_iota(_I32, (nb, rows, cols), 0)
    imax = np.int32(2147483647)

    rk0, rk1 = _threefry2x32(_I32(0), _I32(42), _I32(0), brow)
    zero = jnp.zeros_like(flat_iota)
    round_sort_keys = []
    for _ in range(num_rounds):
        nk0, nk1 = _threefry2x32(rk0, rk1, zero, zero)
        sk0, sk1 = _threefry2x32(rk0, rk1, zero, jnp.ones_like(flat_iota))
        o0, o1 = _threefry2x32(sk0, sk1, zero, flat_iota)
        srt = (o0 ^ o1) ^ _SIGN
        round_sort_keys.append(
            jnp.where(flat_iota < _I32(n_valid), srt,
                      jnp.full_like(flat_iota, imax)))
        rk0, rk1 = nk0, nk1
    k1s = round_sort_keys[0]
    k2s = round_sort_keys[-1]

    v54 = jnp.zeros((nb, 1, 1), _I32)
    for j in range(32):
        bit = np.int32(np.uint32(1 << (31 - j)).astype(np.int32))
        try_pat = v54 | bit
        try_s = try_pat ^ _SIGN
        cnt = jnp.sum(jnp.where(k2s < try_s, _F32(1), _F32(0)),
                      axis=(1, 2), keepdims=True)
        v54 = jnp.where(cnt >= _F32(n_take), v54, try_pat)
    v54_s = v54 ^ _SIGN
    p_ind = jnp.where(k2s <= v54_s, _F32(1), _F32(0))

    _, sidx = _bitonic_sort_pairs(k1s, flat_iota, flat_iota, rows, cols, total)

    ci = jax.lax.broadcasted_iota(_I32, (cols, cols), 0)
    cj = jax.lax.broadcasted_iota(_I32, (cols, cols), 1)
    d_in = cj - ci
    m_in = jnp.where((d_in >= 0) & (d_in < _I32(mask_len)), _F32(1), _F32(0))
    d_x = cj + _I32(cols) - ci
    m_x = jnp.where((d_x >= 0) & (d_x < _I32(mask_len)), _F32(1), _F32(0))
    icol = icol_ref[:, :]
    ih_col = icol[:rows, :]

    for r in range(nb):
        s_flat = sidx[r].reshape(1, total)
        p_flat = p_ind[r].reshape(1, total)
        vh = jax.lax.shift_right_logical(s_flat, _I32(7)).astype(_F32)
        vl = (s_flat & _I32(cols - 1)).astype(_F32)
        s1t = jnp.where((vh == ih_col) & (p_flat > _F32(0)), _F32(1), _F32(0))
        s2t = jnp.where(vl == icol, _F32(1), _F32(0))
        sel = jax.lax.dot_general(
            s1t, s2t, dimension_numbers=(((1,), (1,)), ((), ())),
            preferred_element_type=_F32)
        hit = jax.lax.dot_general(sel, m_in,
                                  dimension_numbers=(((1,), (0,)), ((), ())),
                                  preferred_element_type=_F32)
        sel_prev = jnp.concatenate(
            [jnp.zeros((1, cols), _F32), sel[:rows - 1, :]], axis=0)
        hit = hit + jax.lax.dot_general(
            sel_prev, m_x, dimension_numbers=(((1,), (0,)), ((), ())),
            preferred_element_type=_F32)
        o_ref[r, :, :] = (hit > _F32(0)).astype(jnp.int8)


@jax.jit
def kernel(x):
    B, T, C = x.shape
    total_masked_length = int(T * _MASK_PROB)
    num_masks = math.ceil(total_masked_length / _MASK_LENGTH)
    valid_starts = T - _MASK_LENGTH + 1
    if valid_starts <= 0:
        return jnp.zeros((B, T), dtype=bool)
    n_take = min(num_masks, valid_starts)
    num_rounds = int(
        np.ceil(3 * np.log(max(1, valid_starts)) / np.log(2**32 - 1)))
    cols = 128
    rows = T // cols

    out = pl.pallas_call(
        partial(_mask_kernel, nb=B, rows=rows, cols=cols, n_valid=valid_starts,
                n_take=n_take, num_rounds=num_rounds, mask_len=_MASK_LENGTH),
        grid=(1,),
        in_specs=[pl.BlockSpec((cols, 1), lambda b: (0, 0))],
        out_specs=pl.BlockSpec((B, rows, cols), lambda b: (0, 0, 0)),
        out_shape=jax.ShapeDtypeStruct((B, rows, cols), jnp.int8),
    )(jnp.arange(cols, dtype=jnp.float32).reshape(cols, 1))
    return out.reshape(B, T).astype(bool)
